# Initial kernel scaffold; baseline (speedup 1.0000x reference)
import jax
import jax.numpy as jnp
from jax.experimental import pallas as pl


def kernel(latent):
    raise NotImplementedError("write your pallas kernel here")


# speedup vs baseline: 17.2994x; 17.2994x over previous
import jax
import jax.numpy as jnp
from jax.experimental import pallas as pl
from jax.experimental.pallas import tpu as pltpu

_N = 2048
_D = 768
_R = 256
_K = 5


def _body(x_blk_ref, xt_ref, out_ref, acc_ref):
    i = pl.program_id(0)
    nblk = pl.num_programs(0)

    x_blk = x_blk_ref[...]
    xt = xt_ref[...]

    g = jax.lax.dot_general(
        x_blk, xt, (((1,), (0,)), ((), ())),
        preferred_element_type=jnp.float32,
        precision=jax.lax.Precision.DEFAULT,
    )
    sq_r = jnp.sum(x_blk * x_blk, axis=1, keepdims=True)
    sq_c = jnp.sum(xt * xt, axis=0, keepdims=True)
    d2 = sq_r + sq_c - 2.0 * g

    s = jnp.float32(0.0)
    ss = jnp.float32(0.0)
    for t in range(_K):
        m = jnp.min(d2, axis=1, keepdims=True)
        dist = jnp.sqrt(jnp.maximum(m, 0.0) + 1e-12)
        s = s + jnp.sum(dist)
        ss = ss + jnp.sum(dist * dist)
        if t < _K - 1:
            d2 = jnp.where(d2 <= m, jnp.float32(jnp.inf), d2)

    @pl.when(i == 0)
    def _():
        acc_ref[0] = 0.0
        acc_ref[1] = 0.0

    acc_ref[0] += s
    acc_ref[1] += ss

    @pl.when(i == nblk - 1)
    def _():
        cnt = jnp.float32(_N * _K)
        s1 = acc_ref[0]
        s2 = acc_ref[1]
        var = (s2 - s1 * s1 / cnt) / (cnt - 1.0)
        out_ref[0, 0] = -jnp.sqrt(jnp.maximum(var, 0.0))


def kernel(latent):
    x = latent[0]
    xt = x.T
    out = pl.pallas_call(
        _body,
        grid=(_N // _R,),
        in_specs=[
            pl.BlockSpec((_R, _D), lambda i: (i, 0)),
            pl.BlockSpec((_D, _N), lambda i: (0, 0)),
        ],
        out_specs=pl.BlockSpec((1, 1), lambda i: (0, 0),
                               memory_space=pltpu.SMEM),
        out_shape=jax.ShapeDtypeStruct((1, 1), jnp.float32),
        scratch_shapes=[pltpu.SMEM((2,), jnp.float32)],
    )(x, xt)
    return out[0, 0]
